# baseline (speedup 1.0000x reference)
import jax
import jax.numpy as jnp
from jax.experimental import pallas as pl
from jax.experimental.pallas import tpu as pltpu

_N = 10000
_NFEAT = 128
_NHID = 32
_NCLUSTERS = 10
_ALPHA = 0.2
_HALF = _N // 2
_BR = 200
_NSTEP = _HALF // _BR
_NBUF = 3


def _gc_dec_kernel(x_ref, adjA_ref, adjB_hbm, w_ref, b_ref, mu_ref,
                   h_ref, q_ref, bufB, support_ref, sem):
    r = pl.program_id(0)

    @pl.when(r == 0)
    def _():
        for j in range(_NBUF):
            pltpu.make_async_copy(
                adjB_hbm.at[pl.ds(_HALF + j * _BR, _BR), :],
                bufB.at[j], sem.at[j],
            ).start()
        support_ref[...] = jnp.dot(
            x_ref[...], w_ref[...], preferred_element_type=jnp.float32)

    mu = mu_ref[...]
    mun = jnp.sum(mu * mu, axis=1)[None, :]

    def _finish(h, row0):
        h_ref[pl.ds(row0, _BR), :] = h
        hn = jnp.sum(h * h, axis=1, keepdims=True)
        cross = jnp.dot(h, mu.T, preferred_element_type=jnp.float32)
        dist2 = hn - 2.0 * cross + mun
        q = 1.0 / (1.0 + dist2 / _ALPHA + 1e-08)
        q = q ** (_ALPHA + 1.0) / 2.0
        q_ref[pl.ds(row0, _BR), :] = q / jnp.sum(q, axis=1, keepdims=True)

    hA = jnp.dot(adjA_ref[...], support_ref[...],
                 preferred_element_type=jnp.float32) + b_ref[...]
    _finish(hA, r * _BR)

    slot = r % _NBUF
    pltpu.make_async_copy(
        adjB_hbm.at[pl.ds(_HALF + r * _BR, _BR), :], bufB.at[slot],
        sem.at[slot],
    ).wait()
    hB = jnp.dot(bufB[slot], support_ref[...],
                 preferred_element_type=jnp.float32) + b_ref[...]

    @pl.when(r + _NBUF < _NSTEP)
    def _():
        pltpu.make_async_copy(
            adjB_hbm.at[pl.ds(_HALF + (r + _NBUF) * _BR, _BR), :],
            bufB.at[slot], sem.at[slot],
        ).start()

    _finish(hB, _HALF + r * _BR)


@jax.jit
def kernel(x, adj, W, b, mu):
    h, q = pl.pallas_call(
        _gc_dec_kernel,
        grid=(_NSTEP,),
        in_specs=[
            pl.BlockSpec((_N, _NFEAT), lambda r: (0, 0)),
            pl.BlockSpec((_BR, _N), lambda r: (r, 0)),
            pl.BlockSpec(memory_space=pl.ANY),
            pl.BlockSpec((_NFEAT, _NHID), lambda r: (0, 0)),
            pl.BlockSpec((1, _NHID), lambda r: (0, 0)),
            pl.BlockSpec((_NCLUSTERS, _NHID), lambda r: (0, 0)),
        ],
        out_specs=[
            pl.BlockSpec((_N, _NHID), lambda r: (0, 0)),
            pl.BlockSpec((_N, _NCLUSTERS), lambda r: (0, 0)),
        ],
        out_shape=[
            jax.ShapeDtypeStruct((_N, _NHID), jnp.float32),
            jax.ShapeDtypeStruct((_N, _NCLUSTERS), jnp.float32),
        ],
        scratch_shapes=[
            pltpu.VMEM((_NBUF, _BR, _N), jnp.float32),
            pltpu.VMEM((_N, _NHID), jnp.float32),
            pltpu.SemaphoreType.DMA((_NBUF,)),
        ],
        compiler_params=pltpu.CompilerParams(
            vmem_limit_bytes=100 * 1024 * 1024),
    )(x, adj, adj, W, b.reshape(1, _NHID), mu)
    return h, q


# speedup vs baseline: 1.0690x; 1.0690x over previous
import jax
import jax.numpy as jnp
from jax.experimental import pallas as pl
from jax.experimental.pallas import tpu as pltpu

_N = 10000
_NFEAT = 128
_NHID = 32
_NCLUSTERS = 10
_ALPHA = 0.2
_BR = 400
_NBLK = _N // _BR
_NBUF = 3
_NOUT = 2


def _h_copy(i, h_stage, h_ref, sem_h):
    return pltpu.make_async_copy(
        h_stage.at[i % _NOUT], h_ref.at[pl.ds(i * _BR, _BR), :],
        sem_h.at[i % _NOUT])


def _q_copy(i, q_stage, q_ref, sem_q):
    return pltpu.make_async_copy(
        q_stage.at[i % _NOUT], q_ref.at[pl.ds(i * _BR, _BR), :],
        sem_q.at[i % _NOUT])


def _gc_dec_kernel(x_ref, adj_hbm, w_ref, b_ref, mu_ref, h_ref, q_ref,
                   adj_buf, support_ref, h_stage, q_stage,
                   sem, sem_h, sem_q):
    for j in range(_NBUF):
        pltpu.make_async_copy(
            adj_hbm.at[pl.ds(j * _BR, _BR), :], adj_buf.at[j], sem.at[j],
        ).start()

    support_ref[...] = jnp.dot(
        x_ref[...], w_ref[...], preferred_element_type=jnp.float32)
    mu = mu_ref[...]
    mun = jnp.sum(mu * mu, axis=1)[None, :]

    def body(i, carry):
        slot = i % _NBUF
        ostage = i % _NOUT
        pltpu.make_async_copy(
            adj_hbm.at[pl.ds(i * _BR, _BR), :], adj_buf.at[slot], sem.at[slot],
        ).wait()
        h = jnp.dot(adj_buf[slot], support_ref[...],
                    preferred_element_type=jnp.float32) + b_ref[...]

        @pl.when(i + _NBUF < _NBLK)
        def _():
            pltpu.make_async_copy(
                adj_hbm.at[pl.ds((i + _NBUF) * _BR, _BR), :],
                adj_buf.at[slot], sem.at[slot],
            ).start()

        @pl.when(i >= _NOUT)
        def _():
            _h_copy(i - _NOUT, h_stage, h_ref, sem_h).wait()
            _q_copy(i - _NOUT, q_stage, q_ref, sem_q).wait()

        h_stage[ostage] = h
        _h_copy(i, h_stage, h_ref, sem_h).start()

        hn = jnp.sum(h * h, axis=1, keepdims=True)
        cross = jnp.dot(h, mu.T, preferred_element_type=jnp.float32)
        dist2 = hn - 2.0 * cross + mun
        q = 1.0 / (1.0 + dist2 / _ALPHA + 1e-08)
        q = q ** (_ALPHA + 1.0) / 2.0
        q_stage[ostage] = q / jnp.sum(q, axis=1, keepdims=True)
        _q_copy(i, q_stage, q_ref, sem_q).start()
        return carry

    jax.lax.fori_loop(0, _NBLK, body, 0)
    for i in range(_NBLK - _NOUT, _NBLK):
        _h_copy(i, h_stage, h_ref, sem_h).wait()
        _q_copy(i, q_stage, q_ref, sem_q).wait()


@jax.jit
def kernel(x, adj, W, b, mu):
    h, q = pl.pallas_call(
        _gc_dec_kernel,
        in_specs=[
            pl.BlockSpec(memory_space=pltpu.MemorySpace.VMEM),
            pl.BlockSpec(memory_space=pl.ANY),
            pl.BlockSpec(memory_space=pltpu.MemorySpace.VMEM),
            pl.BlockSpec(memory_space=pltpu.MemorySpace.VMEM),
            pl.BlockSpec(memory_space=pltpu.MemorySpace.VMEM),
        ],
        out_specs=[
            pl.BlockSpec(memory_space=pl.ANY),
            pl.BlockSpec(memory_space=pl.ANY),
        ],
        out_shape=[
            jax.ShapeDtypeStruct((_N, _NHID), jnp.float32),
            jax.ShapeDtypeStruct((_N, _NCLUSTERS), jnp.float32),
        ],
        scratch_shapes=[
            pltpu.VMEM((_NBUF, _BR, _N), jnp.float32),
            pltpu.VMEM((_N, _NHID), jnp.float32),
            pltpu.VMEM((_NOUT, _BR, _NHID), jnp.float32),
            pltpu.VMEM((_NOUT, _BR, _NCLUSTERS), jnp.float32),
            pltpu.SemaphoreType.DMA((_NBUF,)),
            pltpu.SemaphoreType.DMA((_NOUT,)),
            pltpu.SemaphoreType.DMA((_NOUT,)),
        ],
        compiler_params=pltpu.CompilerParams(
            vmem_limit_bytes=100 * 1024 * 1024),
    )(x, adj, W, b.reshape(1, _NHID), mu)
    return h, q
